# baseline (device time: 920823 ns/iter reference)
import jax
import jax.numpy as jnp
from jax import lax
from jax.experimental import pallas as pl
from jax.experimental.pallas import tpu as pltpu

CHUNK_ROWS = [512] * 28 + [128] * 16
N_CHUNKS = len(CHUNK_ROWS)
CHUNK_OFF = [sum(CHUNK_ROWS[:k]) for k in range(N_CHUNKS)]
MAX_UPFRONT = 32

N_LOCAL = 32


def kernel(x):
    m_per, n = x.shape
    m_global = 2 * m_per
    half = m_per // 2
    assert sum(CHUNK_ROWS) == half
    lc = m_per // N_LOCAL

    def body(x_ref, out_ref, stage, in_sems, out_sems,
             x_send, x_recv, y_send, y_recv):
        my_x = lax.axis_index("x")
        my_y = lax.axis_index("y")
        xn = (1 - my_x, my_y)
        yn = (my_x, 1 - my_y)

        barrier_sem = pltpu.get_barrier_semaphore()
        for nbr in (xn, yn):
            pl.semaphore_signal(
                barrier_sem, inc=1, device_id=nbr,
                device_id_type=pl.DeviceIdType.MESH,
            )
        pl.semaphore_wait(barrier_sem, 2)

        def make_x_rdma(k):
            src_row = my_y * half + CHUNK_OFF[k]
            dst_row = my_x * m_per + my_y * half + CHUNK_OFF[k]
            return pltpu.make_async_remote_copy(
                src_ref=x_ref.at[pl.ds(src_row, CHUNK_ROWS[k])],
                dst_ref=out_ref.at[pl.ds(dst_row, CHUNK_ROWS[k])],
                send_sem=x_send.at[k],
                recv_sem=x_recv.at[k],
                device_id=xn,
                device_id_type=pl.DeviceIdType.MESH,
            )

        x_rdmas = []
        for k in range(min(MAX_UPFRONT, N_CHUNKS)):
            r = make_x_rdma(k)
            r.start()
            x_rdmas.append(r)

        out_dmas = []
        y_rdmas = []
        for k in range(N_CHUNKS):
            if MAX_UPFRONT + k < N_CHUNKS:
                r = make_x_rdma(MAX_UPFRONT + k)
                r.start()
                x_rdmas.append(r)

            if k < N_LOCAL:
                slot = k % 2
                if k >= 2:
                    out_dmas[k - 2].wait()
                d_in = pltpu.make_async_copy(
                    x_ref.at[pl.ds(k * lc, lc)], stage.at[slot],
                    in_sems.at[slot],
                )
                d_in.start()
                d_in.wait()
                d_out = pltpu.make_async_copy(
                    stage.at[slot],
                    out_ref.at[pl.ds(my_x * m_per + k * lc, lc)],
                    out_sems.at[slot],
                )
                d_out.start()
                out_dmas.append(d_out)

            x_rdmas[k].wait_recv()
            row = (1 - my_x) * m_per + my_y * half + CHUNK_OFF[k]
            r = pltpu.make_async_remote_copy(
                src_ref=out_ref.at[pl.ds(row, CHUNK_ROWS[k])],
                dst_ref=out_ref.at[pl.ds(row, CHUNK_ROWS[k])],
                send_sem=y_send.at[k],
                recv_sem=y_recv.at[k],
                device_id=yn,
                device_id_type=pl.DeviceIdType.MESH,
            )
            r.start()
            y_rdmas.append(r)

        for k in range(N_CHUNKS):
            x_rdmas[k].wait_send()
            y_rdmas[k].wait_send()
            y_rdmas[k].wait_recv()
        out_dmas[-2].wait()
        out_dmas[-1].wait()

    return pl.pallas_call(
        body,
        out_shape=jax.ShapeDtypeStruct((m_global, n), x.dtype),
        in_specs=[pl.BlockSpec(memory_space=pl.ANY)],
        out_specs=pl.BlockSpec(memory_space=pl.ANY),
        scratch_shapes=[
            pltpu.VMEM((2, m_per // N_LOCAL, n), x.dtype),
            pltpu.SemaphoreType.DMA((2,)),
            pltpu.SemaphoreType.DMA((2,)),
            pltpu.SemaphoreType.DMA((N_CHUNKS,)),
            pltpu.SemaphoreType.DMA((N_CHUNKS,)),
            pltpu.SemaphoreType.DMA((N_CHUNKS,)),
            pltpu.SemaphoreType.DMA((N_CHUNKS,)),
        ],
        compiler_params=pltpu.CompilerParams(collective_id=0),
    )(x)


# device time: 910178 ns/iter; 1.0117x vs baseline; 1.0117x over previous
import jax
import jax.numpy as jnp
from jax import lax
from jax.experimental import pallas as pl
from jax.experimental.pallas import tpu as pltpu

CHUNK_ROWS = [256] * 64
N_CHUNKS = len(CHUNK_ROWS)
CHUNK_OFF = [sum(CHUNK_ROWS[:k]) for k in range(N_CHUNKS)]
MAX_UPFRONT = 32

N_LOCAL = 32


def kernel(x):
    m_per, n = x.shape
    m_global = 2 * m_per
    half = m_per // 2
    assert sum(CHUNK_ROWS) == half
    lc = m_per // N_LOCAL

    def body(x_ref, out_ref, stage, in_sems, out_sems,
             x_send, x_recv, y_send, y_recv):
        my_x = lax.axis_index("x")
        my_y = lax.axis_index("y")
        xn = (1 - my_x, my_y)
        yn = (my_x, 1 - my_y)

        barrier_sem = pltpu.get_barrier_semaphore()
        for nbr in (xn, yn):
            pl.semaphore_signal(
                barrier_sem, inc=1, device_id=nbr,
                device_id_type=pl.DeviceIdType.MESH,
            )
        pl.semaphore_wait(barrier_sem, 2)

        def make_x_rdma(k):
            src_row = my_y * half + CHUNK_OFF[k]
            dst_row = my_x * m_per + my_y * half + CHUNK_OFF[k]
            return pltpu.make_async_remote_copy(
                src_ref=x_ref.at[pl.ds(src_row, CHUNK_ROWS[k])],
                dst_ref=out_ref.at[pl.ds(dst_row, CHUNK_ROWS[k])],
                send_sem=x_send.at[k],
                recv_sem=x_recv.at[k],
                device_id=xn,
                device_id_type=pl.DeviceIdType.MESH,
            )

        x_rdmas = []
        for k in range(min(MAX_UPFRONT, N_CHUNKS)):
            r = make_x_rdma(k)
            r.start()
            x_rdmas.append(r)

        out_dmas = []
        y_rdmas = []
        for k in range(N_CHUNKS):
            if MAX_UPFRONT + k < N_CHUNKS:
                r = make_x_rdma(MAX_UPFRONT + k)
                r.start()
                x_rdmas.append(r)

            if k < N_LOCAL:
                slot = k % 2
                if k >= 2:
                    out_dmas[k - 2].wait()
                d_in = pltpu.make_async_copy(
                    x_ref.at[pl.ds(k * lc, lc)], stage.at[slot],
                    in_sems.at[slot],
                )
                d_in.start()
                d_in.wait()
                d_out = pltpu.make_async_copy(
                    stage.at[slot],
                    out_ref.at[pl.ds(my_x * m_per + k * lc, lc)],
                    out_sems.at[slot],
                )
                d_out.start()
                out_dmas.append(d_out)

            x_rdmas[k].wait_recv()
            row = (1 - my_x) * m_per + my_y * half + CHUNK_OFF[k]
            r = pltpu.make_async_remote_copy(
                src_ref=out_ref.at[pl.ds(row, CHUNK_ROWS[k])],
                dst_ref=out_ref.at[pl.ds(row, CHUNK_ROWS[k])],
                send_sem=y_send.at[k],
                recv_sem=y_recv.at[k],
                device_id=yn,
                device_id_type=pl.DeviceIdType.MESH,
            )
            r.start()
            y_rdmas.append(r)

        for k in range(N_CHUNKS):
            x_rdmas[k].wait_send()
            y_rdmas[k].wait_send()
            y_rdmas[k].wait_recv()
        out_dmas[-2].wait()
        out_dmas[-1].wait()

    return pl.pallas_call(
        body,
        out_shape=jax.ShapeDtypeStruct((m_global, n), x.dtype),
        in_specs=[pl.BlockSpec(memory_space=pl.ANY)],
        out_specs=pl.BlockSpec(memory_space=pl.ANY),
        scratch_shapes=[
            pltpu.VMEM((2, m_per // N_LOCAL, n), x.dtype),
            pltpu.SemaphoreType.DMA((2,)),
            pltpu.SemaphoreType.DMA((2,)),
            pltpu.SemaphoreType.DMA((N_CHUNKS,)),
            pltpu.SemaphoreType.DMA((N_CHUNKS,)),
            pltpu.SemaphoreType.DMA((N_CHUNKS,)),
            pltpu.SemaphoreType.DMA((N_CHUNKS,)),
        ],
        compiler_params=pltpu.CompilerParams(collective_id=0),
    )(x)
